# Initial kernel scaffold; baseline (speedup 1.0000x reference)
import jax
import jax.numpy as jnp
from jax.experimental import pallas as pl


def kernel(x, edge_index, edge_weight, Wq, bq, Wk, bk, Wv, bv):
    raise NotImplementedError("write your pallas kernel here")


# speedup vs baseline: 2.6116x; 2.6116x over previous
import functools
import math

import jax
import jax.numpy as jnp
from jax import lax
from jax.experimental import pallas as pl
from jax.experimental.pallas import tpu as pltpu
from jax.experimental.pallas import tpu_sc as plsc

_NC = 2
_NS = 16
_NW = _NC * _NS
_L = 16

_CHUNK = 80


def _qkv_body(x_ref, wq_ref, wk_ref, wv_ref, bq_ref, bk_ref, bv_ref,
              q_ref, k_ref, v_ref):
    xb = x_ref[...]
    dn = (((1,), (1,)), ((), ()))
    q_ref[...] = lax.dot_general(xb, wq_ref[...], dn,
                                 preferred_element_type=jnp.float32) + bq_ref[...]
    k_ref[...] = lax.dot_general(xb, wk_ref[...], dn,
                                 preferred_element_type=jnp.float32) + bk_ref[...]
    v_ref[...] = lax.dot_general(xb, wv_ref[...], dn,
                                 preferred_element_type=jnp.float32) + bv_ref[...]


def _qkv(x, Wq, Wk, Wv, bq, bk, bv):
    n, d = x.shape
    blk = 2000
    grid = n // blk
    row_spec = pl.BlockSpec((blk, d), lambda i: (i, 0))
    w_spec = pl.BlockSpec((d, d), lambda i: (0, 0))
    b_spec = pl.BlockSpec((1, d), lambda i: (0, 0))
    out = jax.ShapeDtypeStruct((n, d), jnp.float32)
    return pl.pallas_call(
        _qkv_body,
        grid=(grid,),
        in_specs=[row_spec, w_spec, w_spec, w_spec, b_spec, b_spec, b_spec],
        out_specs=[row_spec, row_spec, row_spec],
        out_shape=[out, out, out],
    )(x, Wq, Wk, Wv, bq.reshape(1, d), bk.reshape(1, d), bv.reshape(1, d))


def _make_energy(n, e, d):
    epw = e // _NW
    nchunk = epw // _CHUNK
    inv_scale = 1.0 / math.sqrt(d)
    mesh = plsc.VectorSubcoreMesh(core_axis_name="c", subcore_axis_name="s")

    @functools.partial(
        pl.kernel,
        out_type=jax.ShapeDtypeStruct((e,), jnp.float32),
        mesh=mesh,
        scratch_types=[
            pltpu.VMEM((_CHUNK,), jnp.int32),
            pltpu.VMEM((_CHUNK,), jnp.int32),
            pltpu.VMEM((_CHUNK,), jnp.float32),
            pltpu.VMEM((_CHUNK, d), jnp.float32),
            pltpu.VMEM((_CHUNK, d), jnp.float32),
            pltpu.VMEM((_CHUNK,), jnp.float32),
            pltpu.SemaphoreType.DMA,
            pltpu.SemaphoreType.DMA,
        ],
        compiler_params=pltpu.CompilerParams(needs_layout_passes=False),
    )
    def energy_kernel(q_hbm, k_hbm, row_hbm, col_hbm, ew_hbm, z_hbm,
                      ridx, cidx, ew_v, qrows, krows, zv, semq, semk):
        cid = lax.axis_index("c")
        sid = lax.axis_index("s")
        wid = sid * _NC + cid
        lanes = lax.iota(jnp.int32, _L)

        def chunk_body(ci, carry):
            base = wid * epw + ci * _CHUNK
            pltpu.sync_copy(row_hbm.at[pl.ds(base, _CHUNK)], ridx)
            pltpu.sync_copy(col_hbm.at[pl.ds(base, _CHUNK)], cidx)
            pltpu.sync_copy(ew_hbm.at[pl.ds(base, _CHUNK)], ew_v)
            cq = pltpu.async_copy(q_hbm.at[ridx], qrows, semq)
            ck = pltpu.async_copy(k_hbm.at[cidx], krows, semk)
            cq.wait()
            ck.wait()
            for g in range(_CHUNK // _L):
                ev = jnp.zeros((_L,), jnp.float32)
                for i in range(_L):
                    ei = g * _L + i
                    acc = qrows[ei, pl.ds(0, _L)] * krows[ei, pl.ds(0, _L)]
                    for j in range(1, d // _L):
                        acc = acc + (qrows[ei, pl.ds(j * _L, _L)] *
                                     krows[ei, pl.ds(j * _L, _L)])
                    ev = jnp.where(lanes == i, jnp.sum(acc), ev)
                zv[pl.ds(g * _L, _L)] = (ev * ew_v[pl.ds(g * _L, _L)] *
                                         inv_scale)
            pltpu.sync_copy(zv, z_hbm.at[pl.ds(base, _CHUNK)])
            return carry

        lax.fori_loop(0, nchunk, chunk_body, 0)

    return energy_kernel


def _softmax_body(z_ref, a_ref):
    z = z_ref[...]
    m = jnp.max(z)
    p = jnp.exp(z - m)
    a_ref[...] = p / jnp.sum(p)


def _softmax(z2d):
    return pl.pallas_call(
        _softmax_body,
        out_shape=jax.ShapeDtypeStruct(z2d.shape, jnp.float32),
    )(z2d)


def _make_scatter(n, e, d):
    epw = e // _NW
    nchunk = epw // _CHUNK
    zrows = 200
    ncopy_total = n // zrows
    ncopy_iters = -(-ncopy_total // _NS)
    mesh = plsc.VectorSubcoreMesh(core_axis_name="c", subcore_axis_name="s")

    @functools.partial(
        pl.kernel,
        out_type=jax.ShapeDtypeStruct((_NC, n, d), jnp.float32),
        mesh=mesh,
        scratch_types=[
            pltpu.VMEM((_CHUNK,), jnp.int32),
            pltpu.VMEM((_CHUNK,), jnp.int32),
            pltpu.VMEM((_CHUNK,), jnp.float32),
            pltpu.VMEM((_CHUNK, d), jnp.float32),
            pltpu.VMEM((zrows, d), jnp.float32),
            pltpu.VMEM_SHARED((n, d), jnp.float32),
            pltpu.SemaphoreType.DMA,
        ],
        compiler_params=pltpu.CompilerParams(needs_layout_passes=False),
    )
    def scatter_kernel(v_hbm, row_hbm, col_hbm, attn_hbm, out_hbm,
                       ridx, cidx, a_v, vrows, zbuf, acc_sh, semv):
        cid = lax.axis_index("c")
        sid = lax.axis_index("s")
        wid = sid * _NC + cid

        def zrow_body(r, carry):
            for j in range(d // _L):
                zbuf[r, pl.ds(j * _L, _L)] = jnp.zeros((_L,), jnp.float32)
            return carry

        lax.fori_loop(0, zrows, zrow_body, 0)

        def zcopy_body(t, carry):
            blk = t * _NS + sid

            @pl.when(blk < ncopy_total)
            def _():
                pltpu.sync_copy(zbuf, acc_sh.at[pl.ds(blk * zrows, zrows)])

            return carry

        lax.fori_loop(0, ncopy_iters, zcopy_body, 0)
        plsc.subcore_barrier()

        def chunk_body(ci, carry):
            base = wid * epw + ci * _CHUNK
            pltpu.sync_copy(row_hbm.at[pl.ds(base, _CHUNK)], ridx)
            pltpu.sync_copy(col_hbm.at[pl.ds(base, _CHUNK)], cidx)
            pltpu.sync_copy(attn_hbm.at[pl.ds(base, _CHUNK)], a_v)
            pltpu.async_copy(v_hbm.at[cidx], vrows, semv).wait()
            for g in range(_CHUNK // _L):
                a16 = a_v[pl.ds(g * _L, _L)]
                for i in range(_L):
                    ei = g * _L + i
                    a = a16[i]
                    for j in range(d // _L):
                        vrows[ei, pl.ds(j * _L, _L)] = (
                            vrows[ei, pl.ds(j * _L, _L)] * a)
            pltpu.sync_copy(vrows, acc_sh.at[ridx], add=True)
            return carry

        lax.fori_loop(0, nchunk, chunk_body, 0)
        plsc.subcore_barrier()

        def ocopy_body(t, carry):
            blk = t * _NS + sid

            @pl.when(blk < ncopy_total)
            def _():
                r0 = blk * zrows
                pltpu.sync_copy(acc_sh.at[pl.ds(r0, zrows)],
                                out_hbm.at[cid, pl.ds(r0, zrows)])

            return carry

        lax.fori_loop(0, ncopy_iters, ocopy_body, 0)

    return scatter_kernel


def _combine_body(p_ref, o_ref):
    o_ref[...] = p_ref[0] + p_ref[1]


def _combine(part):
    _, n, d = part.shape
    blk = 2000
    return pl.pallas_call(
        _combine_body,
        grid=(n // blk,),
        in_specs=[pl.BlockSpec((2, blk, d), lambda i: (0, i, 0))],
        out_specs=pl.BlockSpec((blk, d), lambda i: (i, 0)),
        out_shape=jax.ShapeDtypeStruct((n, d), jnp.float32),
    )(part)


def kernel(x, edge_index, edge_weight, Wq, bq, Wk, bk, Wv, bv):
    n, d = x.shape
    e = edge_weight.shape[0]
    row = edge_index[0]
    col = edge_index[1]

    q, k, v = _qkv(x, Wq, Wk, Wv, bq, bk, bv)
    z = _make_energy(n, e, d)(q, k, row, col, edge_weight)
    attn = _softmax(z.reshape(e // 128, 128)).reshape(e)
    part = _make_scatter(n, e, d)(v, row, col, attn)
    return _combine(part)
